# baseline (speedup 1.0000x reference)
import functools

import jax
import jax.numpy as jnp
from jax import lax
from jax.experimental import pallas as pl
from jax.experimental.pallas import tpu as pltpu
from jax.experimental.pallas import tpu_sc as plsc

SEQ = 8192
HID = 4096
N = SEQ * HID
NWORKERS = 32
W = N // NWORKERS
CH = 16384
NCHUNKS = W // CH
NH = NCHUNKS // 2
LANES = 16
UNROLL = 8


def _sc_body(x_hbm, pe_hbm, out_hbm,
             xb0, xb1, pb0, pb1, ob0, ob1,
             sgx0, sgx1, sgp0, sgp1, sso0, sso1):
    cid = lax.axis_index("c")
    sid = lax.axis_index("s")
    base = (sid * 2 + cid) * W

    xb = (xb0, xb1)
    pb = (pb0, pb1)
    ob = (ob0, ob1)
    sgx = (sgx0, sgx1)
    sgp = (sgp0, sgp1)
    sso = (sso0, sso1)

    def gather_start(c, b):
        off = base + c * CH
        pltpu.async_copy(x_hbm.at[pl.ds(off, CH)], xb[b], sgx[b])
        pltpu.async_copy(pe_hbm.at[pl.ds(off, CH)], pb[b], sgp[b])

    def gather_wait(c, b):
        off = base + c * CH
        pltpu.make_async_copy(x_hbm.at[pl.ds(off, CH)], xb[b], sgx[b]).wait()
        pltpu.make_async_copy(pe_hbm.at[pl.ds(off, CH)], pb[b], sgp[b]).wait()

    def scatter_start(c, b):
        off = base + c * CH
        pltpu.async_copy(ob[b], out_hbm.at[pl.ds(off, CH)], sso[b])

    def scatter_wait(c, b):
        off = base + c * CH
        pltpu.make_async_copy(ob[b], out_hbm.at[pl.ds(off, CH)], sso[b]).wait()

    def add_chunk(b):
        xr, pr, orr = xb[b], pb[b], ob[b]

        def body(j, _):
            s = j * (LANES * UNROLL)
            for u in range(UNROLL):
                o = s + u * LANES
                orr[pl.ds(o, LANES)] = (
                    xr[pl.ds(o, LANES)] + pr[pl.ds(o, LANES)]
                )
            return 0

        lax.fori_loop(0, CH // (LANES * UNROLL), body, 0, unroll=False)

    gather_start(0, 0)
    gather_start(1, 1)

    for b in (0, 1):
        gather_wait(b, b)
        add_chunk(b)
        scatter_start(b, b)
        gather_start(2 + b, b)

    def outer(i, _):
        for b in (0, 1):
            c = 2 * i + b
            gather_wait(c, b)
            scatter_wait(c - 2, b)
            add_chunk(b)
            scatter_start(c, b)
            gather_start(c + 2, b)
        return 0

    lax.fori_loop(1, NH - 1, outer, 0, unroll=False)

    for b in (0, 1):
        c = 2 * (NH - 1) + b
        gather_wait(c, b)
        scatter_wait(c - 2, b)
        add_chunk(b)
        scatter_start(c, b)

    for b in (0, 1):
        scatter_wait(2 * (NH - 1) + b, b)


@jax.jit
def _sc_add(x_flat, pe_flat):
    mesh = plsc.VectorSubcoreMesh(core_axis_name="c", subcore_axis_name="s")
    f = functools.partial(
        pl.kernel,
        out_type=jax.ShapeDtypeStruct((N,), jnp.float32),
        mesh=mesh,
        scratch_types=[
            pltpu.VMEM((CH,), jnp.float32),
            pltpu.VMEM((CH,), jnp.float32),
            pltpu.VMEM((CH,), jnp.float32),
            pltpu.VMEM((CH,), jnp.float32),
            pltpu.VMEM((CH,), jnp.float32),
            pltpu.VMEM((CH,), jnp.float32),
            pltpu.SemaphoreType.DMA,
            pltpu.SemaphoreType.DMA,
            pltpu.SemaphoreType.DMA,
            pltpu.SemaphoreType.DMA,
            pltpu.SemaphoreType.DMA,
            pltpu.SemaphoreType.DMA,
        ],
    )(_sc_body)
    return f(x_flat, pe_flat)


def kernel(x, pe_weight):
    out = _sc_add(x.reshape(-1), pe_weight.reshape(-1))
    return out.reshape(x.shape)


# speedup vs baseline: 2.9885x; 2.9885x over previous
import functools

import jax
import jax.numpy as jnp
from jax import lax
from jax.experimental import pallas as pl
from jax.experimental.pallas import tpu as pltpu
from jax.experimental.pallas import tpu_sc as plsc

SEQ = 8192
HID = 4096
NWORKERS = 32
CR = 8
CC = 2048
CHUNKS_PER_ROWBLOCK = HID // CC
RB_PER_WORKER = (SEQ // CR) // NWORKERS
NCHUNKS = RB_PER_WORKER * CHUNKS_PER_ROWBLOCK
NH = NCHUNKS // 2
LANES = 16
UNROLL = 8


def _sc_body(x_hbm, pe_hbm, out_hbm,
             xb0, xb1, pb0, pb1, ob0, ob1,
             sgx0, sgx1, sgp0, sgp1, sso0, sso1):
    cid = lax.axis_index("c")
    sid = lax.axis_index("s")
    wid = sid * 2 + cid
    rb_base = wid * RB_PER_WORKER

    xb = (xb0, xb1)
    pb = (pb0, pb1)
    ob = (ob0, ob1)
    sgx = (sgx0, sgx1)
    sgp = (sgp0, sgp1)
    sso = (sso0, sso1)

    def chunk_slice(ref, c):
        r = (rb_base + c // CHUNKS_PER_ROWBLOCK) * CR
        col = (c % CHUNKS_PER_ROWBLOCK) * CC
        return ref.at[pl.ds(r, CR), pl.ds(col, CC)]

    def gather_start(c, b):
        pltpu.async_copy(chunk_slice(x_hbm, c), xb[b], sgx[b])
        pltpu.async_copy(chunk_slice(pe_hbm, c), pb[b], sgp[b])

    def gather_wait(c, b):
        pltpu.make_async_copy(chunk_slice(x_hbm, c), xb[b], sgx[b]).wait()
        pltpu.make_async_copy(chunk_slice(pe_hbm, c), pb[b], sgp[b]).wait()

    def scatter_start(c, b):
        pltpu.async_copy(ob[b], chunk_slice(out_hbm, c), sso[b])

    def scatter_wait(c, b):
        pltpu.make_async_copy(ob[b], chunk_slice(out_hbm, c), sso[b]).wait()

    def add_chunk(b):
        xr, pr, orr = xb[b], pb[b], ob[b]

        def body(j, _):
            s = j * (LANES * UNROLL)
            for u in range(UNROLL):
                col = s + u * LANES
                for r in range(CR):
                    orr[r, pl.ds(col, LANES)] = (
                        xr[r, pl.ds(col, LANES)] + pr[r, pl.ds(col, LANES)]
                    )
            return 0

        lax.fori_loop(0, CC // (LANES * UNROLL), body, 0, unroll=False)

    gather_start(0, 0)
    gather_start(1, 1)

    for b in (0, 1):
        gather_wait(b, b)
        add_chunk(b)
        scatter_start(b, b)
        gather_start(2 + b, b)

    def outer(i, _):
        for b in (0, 1):
            c = 2 * i + b
            gather_wait(c, b)
            scatter_wait(c - 2, b)
            add_chunk(b)
            scatter_start(c, b)
            gather_start(c + 2, b)
        return 0

    lax.fori_loop(1, NH - 1, outer, 0, unroll=False)

    for b in (0, 1):
        c = 2 * (NH - 1) + b
        gather_wait(c, b)
        scatter_wait(c - 2, b)
        add_chunk(b)
        scatter_start(c, b)

    for b in (0, 1):
        scatter_wait(2 * (NH - 1) + b, b)


@jax.jit
def _sc_add(x, pe):
    mesh = plsc.VectorSubcoreMesh(core_axis_name="c", subcore_axis_name="s")
    f = functools.partial(
        pl.kernel,
        out_type=jax.ShapeDtypeStruct((SEQ, HID), jnp.float32),
        mesh=mesh,
        compiler_params=pltpu.CompilerParams(use_tc_tiling_on_sc=True),
        scratch_types=[
            pltpu.VMEM((CR, CC), jnp.float32),
            pltpu.VMEM((CR, CC), jnp.float32),
            pltpu.VMEM((CR, CC), jnp.float32),
            pltpu.VMEM((CR, CC), jnp.float32),
            pltpu.VMEM((CR, CC), jnp.float32),
            pltpu.VMEM((CR, CC), jnp.float32),
            pltpu.SemaphoreType.DMA,
            pltpu.SemaphoreType.DMA,
            pltpu.SemaphoreType.DMA,
            pltpu.SemaphoreType.DMA,
            pltpu.SemaphoreType.DMA,
            pltpu.SemaphoreType.DMA,
        ],
    )(_sc_body)
    return f(x, pe)


def kernel(x, pe_weight):
    return _sc_add(x, pe_weight)


# speedup vs baseline: 2.9986x; 1.0034x over previous
import functools

import jax
import jax.numpy as jnp
from jax import lax
from jax.experimental import pallas as pl
from jax.experimental.pallas import tpu as pltpu
from jax.experimental.pallas import tpu_sc as plsc

SEQ = 8192
HID = 4096
NWORKERS = 32
CR = 8
CC = 2048
CHUNKS_PER_ROWBLOCK = HID // CC
RB_PER_WORKER = (SEQ // CR) // NWORKERS
NCHUNKS = RB_PER_WORKER * CHUNKS_PER_ROWBLOCK
NSLOTS = 3
LANES = 16
UNROLL = 8


def _sc_body(x_hbm, pe_hbm, out_hbm,
             ob0, ob1, ob2, pb0, pb1, pb2,
             sgx0, sgx1, sgx2, sgp0, sgp1, sgp2, sso0, sso1, sso2):
    cid = lax.axis_index("c")
    sid = lax.axis_index("s")
    wid = sid * 2 + cid
    rb_base = wid * RB_PER_WORKER

    ob = (ob0, ob1, ob2)
    pb = (pb0, pb1, pb2)
    sgx = (sgx0, sgx1, sgx2)
    sgp = (sgp0, sgp1, sgp2)
    sso = (sso0, sso1, sso2)

    def chunk_slice(ref, c):
        r = (rb_base + c // CHUNKS_PER_ROWBLOCK) * CR
        col = (c % CHUNKS_PER_ROWBLOCK) * CC
        return ref.at[pl.ds(r, CR), pl.ds(col, CC)]

    def gather_start(c, b):
        pltpu.async_copy(chunk_slice(x_hbm, c), ob[b], sgx[b])
        pltpu.async_copy(chunk_slice(pe_hbm, c), pb[b], sgp[b])

    def gather_wait(c, b):
        pltpu.make_async_copy(chunk_slice(x_hbm, c), ob[b], sgx[b]).wait()
        pltpu.make_async_copy(chunk_slice(pe_hbm, c), pb[b], sgp[b]).wait()

    def scatter_start(c, b):
        pltpu.async_copy(ob[b], chunk_slice(out_hbm, c), sso[b])

    def scatter_wait(c, b):
        pltpu.make_async_copy(ob[b], chunk_slice(out_hbm, c), sso[b]).wait()

    def add_chunk(b):
        orr, pr = ob[b], pb[b]

        def body(j, _):
            s = j * (LANES * UNROLL)
            for u in range(UNROLL):
                col = s + u * LANES
                for r in range(CR):
                    plsc.addupdate(
                        orr.at[r, pl.ds(col, LANES)],
                        pr[r, pl.ds(col, LANES)],
                    )
            return 0

        lax.fori_loop(0, CC // (LANES * UNROLL), body, 0, unroll=False)

    def step(c, b, b2, *, first_pair=False, tail=False):
        gather_wait(c, b)
        add_chunk(b)
        scatter_start(c, b)
        if not tail:
            if not first_pair:
                scatter_wait(c - 1, b2)
            gather_start(c + 2, b2)

    gather_start(0, 0)
    gather_start(1, 1)

    step(0, 0, 2, first_pair=True)
    step(1, 1, 0)

    def outer(i, _):
        for k in range(NSLOTS):
            c = NSLOTS * i + k + 2
            step(c, (k + 2) % NSLOTS, (k + 4) % NSLOTS)
        return 0

    lax.fori_loop(0, (NCHUNKS - 4) // NSLOTS, outer, 0, unroll=False)

    step(NCHUNKS - 2, (NCHUNKS - 2) % NSLOTS, 0, tail=True)
    step(NCHUNKS - 1, (NCHUNKS - 1) % NSLOTS, 0, tail=True)

    for c in (NCHUNKS - 3, NCHUNKS - 2, NCHUNKS - 1):
        scatter_wait(c, c % NSLOTS)


@jax.jit
def _sc_add(x, pe):
    mesh = plsc.VectorSubcoreMesh(core_axis_name="c", subcore_axis_name="s")
    f = functools.partial(
        pl.kernel,
        out_type=jax.ShapeDtypeStruct((SEQ, HID), jnp.float32),
        mesh=mesh,
        compiler_params=pltpu.CompilerParams(use_tc_tiling_on_sc=True),
        scratch_types=(
            [pltpu.VMEM((CR, CC), jnp.float32)] * 6
            + [pltpu.SemaphoreType.DMA] * 9
        ),
    )(_sc_body)
    return f(x, pe)


def kernel(x, pe_weight):
    return _sc_add(x, pe_weight)
